# baseline (device time: 305055 ns/iter reference)
import jax
import jax.numpy as jnp
from jax import lax
from jax.experimental import pallas as pl
from jax.experimental.pallas import tpu as pltpu

N_DEV = 32


def kernel(x, Wq, K_ext, V_ext, Wo):
    B, Sq_l, E = x.shape
    H4 = Wq.shape[1] // 64
    R = B * Sq_l
    C = Wq.shape[1]

    xb = x.astype(jnp.bfloat16).reshape(R, E)
    wqt = Wq.T.astype(jnp.bfloat16)
    wob = Wo.astype(jnp.bfloat16)
    kt = K_ext.transpose(2, 0, 1, 3).astype(jnp.bfloat16)
    vt = V_ext.transpose(2, 0, 1, 3).astype(jnp.bfloat16)
    Skv = kt.shape[2]

    def body(x_ref, wqt_ref, k_ref, v_ref, wo_ref, out_ref,
             comm_ref, ctx_ref, send_sems, recv_sems):
        my = lax.axis_index("i")
        left = lax.rem(my - 1 + N_DEV, N_DEV)
        right = lax.rem(my + 1, N_DEV)

        barrier_sem = pltpu.get_barrier_semaphore()
        for nbr in (left, right):
            pl.semaphore_signal(
                barrier_sem, inc=1,
                device_id=(nbr,), device_id_type=pl.DeviceIdType.MESH,
            )
        pl.semaphore_wait(barrier_sem, 2)

        comm_ref[0, 0] = wqt_ref[...]
        comm_ref[0, 1] = wo_ref[...]
        out_ref[...] = jnp.zeros((R, E), jnp.float32)

        rb = lax.broadcasted_iota(jnp.int32, (Sq_l, Skv), 0) // 64
        cb = lax.broadcasted_iota(jnp.int32, (Sq_l, Skv), 1) // 64
        mask = rb == cb

        def make_rdma(slot, other):
            return pltpu.make_async_remote_copy(
                src_ref=comm_ref.at[slot],
                dst_ref=comm_ref.at[other],
                send_sem=send_sems.at[slot],
                recv_sem=recv_sems.at[other],
                device_id=(right,),
                device_id_type=pl.DeviceIdType.MESH,
            )

        def step(s, carry):
            slot = lax.rem(s, 2)
            other = 1 - slot
            hb = lax.rem(my - s + 2 * N_DEV, N_DEV)

            @pl.when(s < N_DEV - 1)
            def _():
                make_rdma(slot, other).start()

            wq_c = comm_ref[slot, 0]
            wo_c = comm_ref[slot, 1]
            q = lax.dot_general(
                x_ref[...], wq_c, (((1,), (1,)), ((), ())),
                preferred_element_type=jnp.float32,
            )
            q = (q * 0.125).astype(jnp.bfloat16)
            kh = k_ref[pl.ds(hb * H4, H4)]
            vh = v_ref[pl.ds(hb * H4, H4)]
            for h in range(H4):
                for b in range(B):
                    qbh = q[b * Sq_l:(b + 1) * Sq_l, h * 64:(h + 1) * 64]
                    sc = lax.dot_general(
                        qbh, kh[h, b], (((1,), (1,)), ((), ())),
                        preferred_element_type=jnp.float32,
                    )
                    sc = jnp.where(mask, sc, -1e9)
                    m = jnp.max(sc, axis=-1, keepdims=True)
                    w = jnp.exp(sc - m)
                    w = (w / jnp.sum(w, axis=-1, keepdims=True)).astype(jnp.bfloat16)
                    ctx = lax.dot_general(
                        w, vh[h, b], (((1,), (0,)), ((), ())),
                        preferred_element_type=jnp.float32,
                    )
                    ctx_ref[b * Sq_l:(b + 1) * Sq_l, h * 64:(h + 1) * 64] = (
                        ctx.astype(jnp.bfloat16))
            out_ref[...] += lax.dot_general(
                ctx_ref[...], wo_c, (((1,), (0,)), ((), ())),
                preferred_element_type=jnp.float32,
            )

            @pl.when(s < N_DEV - 1)
            def _():
                make_rdma(slot, other).wait()

            return carry

        lax.fori_loop(0, N_DEV, step, 0)

    out = pl.pallas_call(
        body,
        out_shape=jax.ShapeDtypeStruct((R, E), jnp.float32),
        in_specs=[pl.BlockSpec(memory_space=pltpu.VMEM)] * 5,
        out_specs=pl.BlockSpec(memory_space=pltpu.VMEM),
        scratch_shapes=[
            pltpu.VMEM((2, 2, C, E), jnp.bfloat16),
            pltpu.VMEM((R, C), jnp.bfloat16),
            pltpu.SemaphoreType.DMA((2,)),
            pltpu.SemaphoreType.DMA((2,)),
        ],
        compiler_params=pltpu.CompilerParams(collective_id=0),
    )(xb, wqt, kt, vt, wob)
    return out.reshape(B, Sq_l, E)


# device time: 275731 ns/iter; 1.1064x vs baseline; 1.1064x over previous
import jax
import jax.numpy as jnp
from jax import lax
from jax.experimental import pallas as pl
from jax.experimental.pallas import tpu as pltpu

N_DEV = 32
HALF = N_DEV // 2


def kernel(x, Wq, K_ext, V_ext, Wo):
    B, Sq_l, E = x.shape
    H4 = Wq.shape[1] // 64
    R = B * Sq_l
    C = Wq.shape[1]

    xb = x.astype(jnp.bfloat16).reshape(R, E)
    wqt = Wq.T.astype(jnp.bfloat16)
    wob = Wo.astype(jnp.bfloat16)
    kt = K_ext.transpose(2, 0, 1, 3).astype(jnp.bfloat16)
    vt = V_ext.transpose(2, 0, 1, 3).astype(jnp.bfloat16)
    Skv = kt.shape[2]

    def body(x_ref, wqt_ref, k_ref, v_ref, wo_ref, out_ref,
             comm_f, comm_b, ctx_ref,
             send_f, recv_f, send_b, recv_b):
        my = lax.axis_index("i")
        left = lax.rem(my - 1 + N_DEV, N_DEV)
        right = lax.rem(my + 1, N_DEV)

        barrier_sem = pltpu.get_barrier_semaphore()
        for nbr in (left, right):
            pl.semaphore_signal(
                barrier_sem, inc=1,
                device_id=(nbr,), device_id_type=pl.DeviceIdType.MESH,
            )
        pl.semaphore_wait(barrier_sem, 2)

        rb = lax.broadcasted_iota(jnp.int32, (Sq_l, Skv), 0) // 64
        cb = lax.broadcasted_iota(jnp.int32, (Sq_l, Skv), 1) // 64
        mask = rb == cb

        def fwd_rdma(slot, other):
            return pltpu.make_async_remote_copy(
                src_ref=comm_f.at[slot], dst_ref=comm_f.at[other],
                send_sem=send_f.at[slot], recv_sem=recv_f.at[other],
                device_id=(right,), device_id_type=pl.DeviceIdType.MESH,
            )

        def bwd_rdma(src_ref, slot, other):
            return pltpu.make_async_remote_copy(
                src_ref=src_ref, dst_ref=comm_b.at[other],
                send_sem=send_b.at[slot], recv_sem=recv_b.at[other],
                device_id=(left,), device_id_type=pl.DeviceIdType.MESH,
            )

        def compute_chunk(wq_c, wo_c, hb):
            q = lax.dot_general(
                x_ref[...], wq_c, (((1,), (1,)), ((), ())),
                preferred_element_type=jnp.float32,
            )
            q = (q * 0.125).astype(jnp.bfloat16)
            kh = k_ref[pl.ds(hb * H4, H4)]
            vh = v_ref[pl.ds(hb * H4, H4)]
            for h in range(H4):
                for b in range(B):
                    qbh = q[b * Sq_l:(b + 1) * Sq_l, h * 64:(h + 1) * 64]
                    sc = lax.dot_general(
                        qbh, kh[h, b], (((1,), (1,)), ((), ())),
                        preferred_element_type=jnp.float32,
                    )
                    sc = jnp.where(mask, sc, -1e9)
                    m = jnp.max(sc, axis=-1, keepdims=True)
                    w = jnp.exp(sc - m)
                    w = (w / jnp.sum(w, axis=-1, keepdims=True)
                         ).astype(jnp.bfloat16)
                    ctx = lax.dot_general(
                        w, vh[h, b], (((1,), (0,)), ((), ())),
                        preferred_element_type=jnp.float32,
                    )
                    ctx_ref[b * Sq_l:(b + 1) * Sq_l, h * 64:(h + 1) * 64] = (
                        ctx.astype(jnp.bfloat16))
            out_ref[...] += lax.dot_general(
                ctx_ref[...], wo_c, (((1,), (0,)), ((), ())),
                preferred_element_type=jnp.float32,
            )

        comm_f[0, 0] = wqt_ref[...]
        comm_f[0, 1] = wo_ref[...]
        out_ref[...] = jnp.zeros((R, E), jnp.float32)

        fwd_rdma(0, 1).start()
        bwd_rdma(comm_f.at[0], 0, 1).start()
        compute_chunk(comm_f[0, 0], comm_f[0, 1], my)
        fwd_rdma(0, 1).wait()
        bwd_rdma(comm_f.at[0], 0, 1).wait()

        def step(s, carry):
            slot = lax.rem(s, 2)
            other = 1 - slot

            fwd_rdma(slot, other).start()

            @pl.when(s < HALF - 1)
            def _():
                bwd_rdma(comm_b.at[slot], slot, other).start()

            compute_chunk(comm_f[slot, 0], comm_f[slot, 1],
                          lax.rem(my - s + N_DEV, N_DEV))
            compute_chunk(comm_b[slot, 0], comm_b[slot, 1],
                          lax.rem(my + s, N_DEV))

            fwd_rdma(slot, other).wait()

            @pl.when(s < HALF - 1)
            def _():
                bwd_rdma(comm_b.at[slot], slot, other).wait()

            return carry

        lax.fori_loop(1, HALF, step, 0)

        compute_chunk(comm_f[0, 0], comm_f[0, 1],
                      lax.rem(my + HALF, N_DEV))

    out = pl.pallas_call(
        body,
        out_shape=jax.ShapeDtypeStruct((R, E), jnp.float32),
        in_specs=[pl.BlockSpec(memory_space=pltpu.VMEM)] * 5,
        out_specs=pl.BlockSpec(memory_space=pltpu.VMEM),
        scratch_shapes=[
            pltpu.VMEM((2, 2, C, E), jnp.bfloat16),
            pltpu.VMEM((2, 2, C, E), jnp.bfloat16),
            pltpu.VMEM((R, C), jnp.bfloat16),
            pltpu.SemaphoreType.DMA((2,)),
            pltpu.SemaphoreType.DMA((2,)),
            pltpu.SemaphoreType.DMA((2,)),
            pltpu.SemaphoreType.DMA((2,)),
        ],
        compiler_params=pltpu.CompilerParams(collective_id=0),
    )(xb, wqt, kt, vt, wob)
    return out.reshape(B, Sq_l, E)
